# Initial kernel scaffold; baseline (speedup 1.0000x reference)
import jax
import jax.numpy as jnp
from jax.experimental import pallas as pl


def kernel(x):
    raise NotImplementedError("write your pallas kernel here")


# speedup vs baseline: 4.8730x; 4.8730x over previous
import functools

import jax
import jax.numpy as jnp
from jax import lax
from jax.experimental import pallas as pl
from jax.experimental.pallas import tpu as pltpu
from jax.experimental.pallas import tpu_sc as plsc

_LANES = 16
_ATOL = 1e-6


def _row_lane_max(buf, l, nvec):
    pm = jnp.abs(buf[l, pl.ds(0, _LANES)])
    for j in range(1, nvec):
        pm = jnp.maximum(pm, jnp.abs(buf[l, pl.ds(j * _LANES, _LANES)]))
    return pm


def _lane_fold_max(v):
    vr = jnp.maximum(v, lax.rev(v, (0,)))
    s = vr[0]
    for k in range(1, 8):
        s = jnp.maximum(s, vr[k])
    return s


def _ffill_body(seq_per_w, L, D, x_hbm, out_hbm, buf):
    nc = plsc.get_sparse_core_info().num_cores
    wid = lax.axis_index("s") * nc + lax.axis_index("c")
    base = wid * seq_per_w
    nvec = D // _LANES
    atol = jnp.float32(_ATOL)

    def process_seq(i, carry):
        s = base + i
        pltpu.sync_copy(x_hbm.at[s], buf)

        def acc_row(l, acc):
            return jnp.minimum(acc, _row_lane_max(buf, l, nvec))

        acc = lax.fori_loop(
            0, L, acc_row, jnp.full((_LANES,), jnp.inf, jnp.float32)
        )

        @pl.when(_lane_fold_max(acc) <= atol)
        def _slow_path():
            def row_loop(l, lv):
                missing = _lane_fold_max(_row_lane_max(buf, l, nvec)) <= atol

                @pl.when(missing)
                def _patch():
                    for j in range(nvec):
                        buf[l, pl.ds(j * _LANES, _LANES)] = (
                            buf[lv, pl.ds(j * _LANES, _LANES)]
                        )

                return jnp.where(missing, lv, l)

            lax.fori_loop(1, L, row_loop, jnp.int32(0))

        pltpu.sync_copy(buf, out_hbm.at[s])
        return carry

    lax.fori_loop(0, seq_per_w, process_seq, jnp.int32(0))


def kernel(x):
    B, L, D = x.shape
    info = plsc.get_sparse_core_info()
    nw = info.num_cores * info.num_subcores
    assert B % nw == 0 and D % _LANES == 0
    seq_per_w = B // nw
    mesh = plsc.VectorSubcoreMesh(core_axis_name="c", subcore_axis_name="s")
    f = pl.kernel(
        functools.partial(_ffill_body, seq_per_w, L, D),
        mesh=mesh,
        out_type=jax.ShapeDtypeStruct((B, L, D), x.dtype),
        scratch_types=[pltpu.VMEM((L, D), jnp.float32)],
    )
    return f(x)


# speedup vs baseline: 8.0926x; 1.6607x over previous
import functools

import jax
import jax.numpy as jnp
from jax import lax
from jax.experimental import pallas as pl
from jax.experimental.pallas import tpu as pltpu
from jax.experimental.pallas import tpu_sc as plsc

_LANES = 16
_ATOL = 1e-6
_NBUF = 4


def _row_lane_max(buf, l, nvec):
    pm = jnp.abs(buf[l, pl.ds(0, _LANES)])
    for j in range(1, nvec):
        pm = jnp.maximum(pm, jnp.abs(buf[l, pl.ds(j * _LANES, _LANES)]))
    return pm


def _lane_fold_max(v):
    vr = jnp.maximum(v, lax.rev(v, (0,)))
    s = vr[0]
    for k in range(1, 8):
        s = jnp.maximum(s, vr[k])
    return s


def _impute_block(buf, L, nvec, atol):
    def acc_row(l, acc):
        return jnp.minimum(acc, _row_lane_max(buf, l, nvec))

    acc = lax.fori_loop(0, L, acc_row, jnp.full((_LANES,), jnp.inf, jnp.float32))

    @pl.when(_lane_fold_max(acc) <= atol)
    def _slow_path():
        def row_loop(l, lv):
            missing = _lane_fold_max(_row_lane_max(buf, l, nvec)) <= atol

            @pl.when(missing)
            def _patch():
                for j in range(nvec):
                    buf[l, pl.ds(j * _LANES, _LANES)] = (
                        buf[lv, pl.ds(j * _LANES, _LANES)]
                    )

            return jnp.where(missing, lv, l)

        lax.fori_loop(1, L, row_loop, jnp.int32(0))


def _ffill_body(seq_per_w, L, D, x_hbm, out_hbm, *refs):
    bufs = refs[:_NBUF]
    sin = refs[_NBUF:2 * _NBUF]
    sout = refs[2 * _NBUF:3 * _NBUF]
    nc = plsc.get_sparse_core_info().num_cores
    wid = lax.axis_index("s") * nc + lax.axis_index("c")
    base = wid * seq_per_w
    nvec = D // _LANES
    atol = jnp.float32(_ATOL)

    for k in range(_NBUF - 1):
        pltpu.make_async_copy(x_hbm.at[base + k], bufs[k], sin[k]).start()

    def chunk(h, carry):
        i0 = h * _NBUF
        for p in range(_NBUF):
            i = i0 + p
            s = base + i
            pltpu.make_async_copy(x_hbm.at[s], bufs[p], sin[p]).wait()
            _impute_block(bufs[p], L, nvec, atol)
            pltpu.make_async_copy(bufs[p], out_hbm.at[s], sout[p]).start()
            q = (p + _NBUF - 1) % _NBUF

            @pl.when(i + _NBUF - 1 < seq_per_w)
            def _prefetch():
                @pl.when(i >= 1)
                def _drain():
                    pltpu.make_async_copy(
                        bufs[q], out_hbm.at[s - 1], sout[q]
                    ).wait()

                pltpu.make_async_copy(
                    x_hbm.at[s + _NBUF - 1], bufs[q], sin[q]
                ).start()

        return carry

    lax.fori_loop(0, seq_per_w // _NBUF, chunk, jnp.int32(0))
    for p in range(_NBUF):
        s_last = base + seq_per_w - _NBUF + p
        pltpu.make_async_copy(bufs[p], out_hbm.at[s_last], sout[p]).wait()


def kernel(x):
    B, L, D = x.shape
    info = plsc.get_sparse_core_info()
    nw = info.num_cores * info.num_subcores
    assert B % (nw * _NBUF) == 0 and D % _LANES == 0
    seq_per_w = B // nw
    mesh = plsc.VectorSubcoreMesh(core_axis_name="c", subcore_axis_name="s")
    f = pl.kernel(
        functools.partial(_ffill_body, seq_per_w, L, D),
        mesh=mesh,
        out_type=jax.ShapeDtypeStruct((B, L, D), x.dtype),
        scratch_types=(
            [pltpu.VMEM((L, D), jnp.float32) for _ in range(_NBUF)]
            + [pltpu.SemaphoreType.DMA for _ in range(2 * _NBUF)]
        ),
    )
    return f(x)


# speedup vs baseline: 8.1219x; 1.0036x over previous
import functools

import jax
import jax.numpy as jnp
from jax import lax
from jax.experimental import pallas as pl
from jax.experimental.pallas import tpu as pltpu
from jax.experimental.pallas import tpu_sc as plsc

_LANES = 16
_ATOL = 1e-6
_NBUF = 4


def _row_lane_max(buf, l, nvec):
    pm = jnp.abs(buf[l, pl.ds(0, _LANES)])
    for j in range(1, nvec):
        pm = jnp.maximum(pm, jnp.abs(buf[l, pl.ds(j * _LANES, _LANES)]))
    return pm


def _lane_fold_max(v):
    vr = jnp.maximum(v, lax.rev(v, (0,)))
    s = vr[0]
    for k in range(1, 8):
        s = jnp.maximum(s, vr[k])
    return s


def _impute_block(buf, L, nvec, atol):
    unroll = 4
    assert L % unroll == 0

    def acc_rows(h, acc):
        l0 = h * unroll
        for d in range(unroll):
            acc = jnp.minimum(acc, jnp.abs(buf[l0 + d, pl.ds(0, _LANES)]))
        return acc

    acc = lax.fori_loop(
        0, L // unroll, acc_rows, jnp.full((_LANES,), jnp.inf, jnp.float32)
    )

    @pl.when(_lane_fold_max(acc) <= atol)
    def _slow_path():
        def row_loop(l, lv):
            missing = _lane_fold_max(_row_lane_max(buf, l, nvec)) <= atol

            @pl.when(missing)
            def _patch():
                for j in range(nvec):
                    buf[l, pl.ds(j * _LANES, _LANES)] = (
                        buf[lv, pl.ds(j * _LANES, _LANES)]
                    )

            return jnp.where(missing, lv, l)

        lax.fori_loop(1, L, row_loop, jnp.int32(0))


def _ffill_body(seq_per_w, L, D, x_hbm, out_hbm, *refs):
    bufs = refs[:_NBUF]
    sin = refs[_NBUF:2 * _NBUF]
    sout = refs[2 * _NBUF:3 * _NBUF]
    nc = plsc.get_sparse_core_info().num_cores
    wid = lax.axis_index("s") * nc + lax.axis_index("c")
    base = wid * seq_per_w
    nvec = D // _LANES
    atol = jnp.float32(_ATOL)

    for k in range(_NBUF - 1):
        pltpu.make_async_copy(x_hbm.at[base + k], bufs[k], sin[k]).start()

    def chunk(h, carry):
        i0 = h * _NBUF
        for p in range(_NBUF):
            i = i0 + p
            s = base + i
            pltpu.make_async_copy(x_hbm.at[s], bufs[p], sin[p]).wait()
            _impute_block(bufs[p], L, nvec, atol)
            pltpu.make_async_copy(bufs[p], out_hbm.at[s], sout[p]).start()
            q = (p + _NBUF - 1) % _NBUF

            @pl.when(i + _NBUF - 1 < seq_per_w)
            def _prefetch():
                @pl.when(i >= 1)
                def _drain():
                    pltpu.make_async_copy(
                        bufs[q], out_hbm.at[s - 1], sout[q]
                    ).wait()

                pltpu.make_async_copy(
                    x_hbm.at[s + _NBUF - 1], bufs[q], sin[q]
                ).start()

        return carry

    lax.fori_loop(0, seq_per_w // _NBUF, chunk, jnp.int32(0))
    for p in range(_NBUF):
        s_last = base + seq_per_w - _NBUF + p
        pltpu.make_async_copy(bufs[p], out_hbm.at[s_last], sout[p]).wait()


def kernel(x):
    B, L, D = x.shape
    info = plsc.get_sparse_core_info()
    nw = info.num_cores * info.num_subcores
    assert B % (nw * _NBUF) == 0 and D % _LANES == 0
    seq_per_w = B // nw
    mesh = plsc.VectorSubcoreMesh(core_axis_name="c", subcore_axis_name="s")
    f = pl.kernel(
        functools.partial(_ffill_body, seq_per_w, L, D),
        mesh=mesh,
        out_type=jax.ShapeDtypeStruct((B, L, D), x.dtype),
        scratch_types=(
            [pltpu.VMEM((L, D), jnp.float32) for _ in range(_NBUF)]
            + [pltpu.SemaphoreType.DMA for _ in range(2 * _NBUF)]
        ),
    )
    return f(x)
